# Initial kernel scaffold; baseline (speedup 1.0000x reference)
import jax
import jax.numpy as jnp
from jax.experimental import pallas as pl


def kernel(x, edge_index, Wl, bl, Wr, Wa, ba, W1, b1, W2, b2, W3, b3):
    raise NotImplementedError("write your pallas kernel here")


# speedup vs baseline: 5.1260x; 5.1260x over previous
import functools

import jax
import jax.numpy as jnp
from jax import lax
from jax.experimental import pallas as pl
from jax.experimental.pallas import tpu as pltpu
from jax.experimental.pallas import tpu_sc as plsc

N = 10000
E = 160000
FH = 128
NC = 2
NT = 16
CHUNK = 128
CPT = 80
EPAD = NT * CPT * CHUNK
NPAD = 10240
STRIPE = NPAD // NT

BM = 1000
CB_M = 1000
CB_N = 2048


def _dot(a, b, precision=None):
    return jnp.dot(a, b, preferred_element_type=jnp.float32,
                   precision=precision)


def _pre_body(x_ref, wlT_ref, wrT_ref, p_ref, r_ref):
    xb = x_ref[...]
    y = _dot(xb, wlT_ref[...], precision=lax.Precision.HIGHEST)
    p_ref[0] = y[:, :FH]
    p_ref[1] = y[:, FH:]
    r_ref[...] = _dot(xb, wrT_ref[...])


def _pre(x, wlT, wrT):
    return pl.pallas_call(
        _pre_body,
        grid=(N // BM,),
        in_specs=[
            pl.BlockSpec((BM, 512), lambda i: (i, 0)),
            pl.BlockSpec((512, 256), lambda i: (0, 0)),
            pl.BlockSpec((512, 256), lambda i: (0, 0)),
        ],
        out_specs=[
            pl.BlockSpec((NC, BM, FH), lambda i: (0, i, 0)),
            pl.BlockSpec((BM, 256), lambda i: (i, 0)),
        ],
        out_shape=[
            jax.ShapeDtypeStruct((NC, N, FH), jnp.float32),
            jax.ShapeDtypeStruct((N, 256), jnp.float32),
        ],
    )(x, wlT, wrT)


def _sc_segment_sum(pflat, src2, dst, z128, ones_h):
    mesh = plsc.VectorSubcoreMesh(core_axis_name="c", subcore_axis_name="s")

    @functools.partial(
        pl.kernel,
        mesh=mesh,
        out_type=jax.ShapeDtypeStruct((NC, NPAD, FH), jnp.float32),
        scratch_types=(
            pltpu.VMEM((CPT, CHUNK), jnp.int32),
            pltpu.VMEM((CPT, CHUNK), jnp.int32),
            pltpu.VMEM((CHUNK, FH), jnp.float32),
            pltpu.VMEM_SHARED((NPAD, FH), jnp.float32),
        ),
    )
    def k(p_hbm, src_hbm, dst_hbm, z128_hbm,
          out_acc, idx_src, idx_dst, buf, acc):
        c = lax.axis_index("c")
        s = lax.axis_index("s")
        pltpu.sync_copy(z128_hbm, acc.at[pl.ds(s * STRIPE, STRIPE)])
        pltpu.sync_copy(src_hbm.at[c].at[s], idx_src)
        pltpu.sync_copy(dst_hbm.at[s], idx_dst)
        plsc.subcore_barrier()

        def body(j, carry):
            pltpu.sync_copy(p_hbm.at[idx_src.at[j]], buf)
            pltpu.sync_copy(buf, acc.at[idx_dst.at[j]], add=True)
            return carry

        lax.fori_loop(0, CPT, body, 0)
        plsc.subcore_barrier()
        sl = pl.ds(s * STRIPE, STRIPE)
        pltpu.sync_copy(acc.at[sl], out_acc.at[c].at[sl])

    @functools.partial(
        pl.kernel,
        mesh=mesh,
        out_type=jax.ShapeDtypeStruct((NC, NPAD, FH), jnp.float32),
        scratch_types=(
            pltpu.VMEM((CPT, CHUNK), jnp.int32),
            pltpu.VMEM((CHUNK, FH), jnp.float32),
            pltpu.VMEM_SHARED((NPAD, FH), jnp.float32),
        ),
    )
    def kc(dst_hbm, z128_hbm, ones_hbm, out_cnt, idx_dst, ones_v, cnt):
        c = lax.axis_index("c")
        s = lax.axis_index("s")
        pltpu.sync_copy(z128_hbm, cnt.at[pl.ds(s * STRIPE, STRIPE)])
        pltpu.sync_copy(dst_hbm.at[s], idx_dst)
        pltpu.sync_copy(ones_hbm, ones_v)
        plsc.subcore_barrier()
        half = CPT // NC

        def body(j, carry):
            pltpu.sync_copy(ones_v, cnt.at[idx_dst.at[j]], add=True)
            return carry

        lax.fori_loop(c * half, (c + 1) * half, body, 0)
        plsc.subcore_barrier()
        sl = pl.ds(s * STRIPE, STRIPE)
        pltpu.sync_copy(cnt.at[sl], out_cnt.at[c].at[sl])

    return k(pflat, src2, dst, z128), kc(dst, z128, ones_h)


def _mlp_body(acc_ref, cnt_ref, r_ref, bl_ref, waT_ref, ba_ref, w1T_ref,
              b1_ref, w2T_ref, b2_ref, w3T_ref, b3_ref, u_ref, v_ref):
    inv = 1.0 / jnp.maximum(cnt_ref[0, :, 0:1] + cnt_ref[1, :, 0:1], 1.0)
    h = jnp.concatenate([acc_ref[0] * inv, acc_ref[1] * inv], axis=1)
    h = jnp.maximum(h + bl_ref[...] + r_ref[...], 0.0)
    h = jnp.maximum(_dot(h, waT_ref[...]) + ba_ref[...], 0.0)
    h = jnp.maximum(_dot(h, w1T_ref[...]) + b1_ref[...], 0.0)
    h = jnp.maximum(_dot(h, w2T_ref[...]) + b2_ref[...], 0.0)
    z = _dot(h, w3T_ref[...]) + b3_ref[...]
    zc = z[:, :3]
    sq = jnp.sum(zc * zc, axis=1, keepdims=True)
    zh = zc.astype(jnp.bfloat16).astype(jnp.float32)
    zl = zc - zh
    sqh = sq.astype(jnp.bfloat16).astype(jnp.float32)
    sql = sq - sqh
    one = jnp.ones_like(sq)
    zero3 = jnp.zeros_like(zc)
    u_ref[...] = jnp.concatenate(
        [zh, zh, zl, sqh, sql, one, one, zero3], axis=1)
    v_ref[...] = jnp.concatenate(
        [-2.0 * zh, -2.0 * zl, -2.0 * zh, one, one, sqh, sql, zero3], axis=1)


def _mlp(acc, cnt, r, blv, waT, bav, w1T, b1v, w2T, b2v, w3T8, b3v):
    full = lambda shape: pl.BlockSpec(shape, lambda i: tuple(0 for _ in shape))
    return pl.pallas_call(
        _mlp_body,
        grid=(N // BM,),
        in_specs=[
            pl.BlockSpec((NC, BM, FH), lambda i: (0, i, 0)),
            pl.BlockSpec((NC, BM, FH), lambda i: (0, i, 0)),
            pl.BlockSpec((BM, 256), lambda i: (i, 0)),
            full((1, 256)),
            full((256, 128)), full((1, 128)),
            full((128, 64)), full((1, 64)),
            full((64, 32)), full((1, 32)),
            full((32, 8)), full((1, 8)),
        ],
        out_specs=[
            pl.BlockSpec((BM, 16), lambda i: (i, 0)),
            pl.BlockSpec((BM, 16), lambda i: (i, 0)),
        ],
        out_shape=[
            jax.ShapeDtypeStruct((N, 16), jnp.float32),
            jax.ShapeDtypeStruct((N, 16), jnp.float32),
        ],
    )(acc, cnt, r, blv, waT, bav, w1T, b1v, w2T, b2v, w3T8, b3v)


def _cdist_body(u_ref, v_ref, o_ref):
    d2 = lax.dot_general(u_ref[...], v_ref[...],
                         (((1,), (1,)), ((), ())),
                         preferred_element_type=jnp.float32)
    o_ref[...] = jnp.sqrt(jnp.maximum(d2, 1e-24))


def _cdist(u, v):
    return pl.pallas_call(
        _cdist_body,
        grid=(N // CB_M, pl.cdiv(N, CB_N)),
        in_specs=[
            pl.BlockSpec((CB_M, 16), lambda i, j: (i, 0)),
            pl.BlockSpec((CB_N, 16), lambda i, j: (j, 0)),
        ],
        out_specs=pl.BlockSpec((CB_M, CB_N), lambda i, j: (i, j)),
        out_shape=jax.ShapeDtypeStruct((N, N), jnp.float32),
    )(u, v)


def kernel(x, edge_index, Wl, bl, Wr, Wa, ba, W1, b1, W2, b2, W3, b3):
    ei = edge_index.astype(jnp.int32)
    src = ei[0]
    dst = ei[1]
    npd = EPAD - E
    pad_src = jnp.arange(npd, dtype=jnp.int32) % N
    pad_dst = N + (jnp.arange(npd, dtype=jnp.int32) % (NPAD - N))
    srcp = jnp.concatenate([src, pad_src]).reshape(NT, CPT, CHUNK)
    dstp = jnp.concatenate([dst, pad_dst]).reshape(NT, CPT, CHUNK)
    src2 = jnp.stack([srcp, srcp + N])

    p, r = _pre(x, Wl.T, Wr.T)
    pflat = p.reshape(NC * N, FH)

    z128 = jnp.zeros((STRIPE, FH), jnp.float32)
    ones_h = jnp.ones((CHUNK, FH), jnp.float32)
    acc, cnt = _sc_segment_sum(pflat, src2, dstp, z128, ones_h)

    blv = bl.reshape(1, 256)
    bav = ba.reshape(1, 128)
    b1v = b1.reshape(1, 64)
    b2v = b2.reshape(1, 32)
    w3T8 = jnp.concatenate([W3.T, jnp.zeros((32, 5), jnp.float32)], axis=1)
    b3v = jnp.concatenate([b3, jnp.zeros((5,), jnp.float32)]).reshape(1, 8)
    u, v = _mlp(acc, cnt, r,
                blv, Wa.T, bav, W1.T, b1v, W2.T, b2v, w3T8, b3v)
    return _cdist(u, v)


# speedup vs baseline: 5.7424x; 1.1202x over previous
import functools

import jax
import jax.numpy as jnp
from jax import lax
from jax.experimental import pallas as pl
from jax.experimental.pallas import tpu as pltpu
from jax.experimental.pallas import tpu_sc as plsc

N = 10000
E = 160000
FH = 128
NC = 2
NT = 16
CHUNK = 128
CPT = 80
G = 16
NG = CPT // G
EPAD = NT * CPT * CHUNK
NPAD = 10240
STRIPE = NPAD // NT

BM = 1000
CB_M = 1000
CB_N = 2048


def _dot(a, b, precision=None):
    return jnp.dot(a, b, preferred_element_type=jnp.float32,
                   precision=precision)


def _pre_body(x_ref, wlT_ref, wrT_ref, p_ref, r_ref):
    xb = x_ref[...]
    y = _dot(xb, wlT_ref[...], precision=lax.Precision.HIGHEST)
    p_ref[0] = y[:, :FH]
    p_ref[1] = y[:, FH:]
    r_ref[...] = _dot(xb, wrT_ref[...])


def _pre(x, wlT, wrT):
    return pl.pallas_call(
        _pre_body,
        grid=(N // BM,),
        in_specs=[
            pl.BlockSpec((BM, 512), lambda i: (i, 0)),
            pl.BlockSpec((512, 256), lambda i: (0, 0)),
            pl.BlockSpec((512, 256), lambda i: (0, 0)),
        ],
        out_specs=[
            pl.BlockSpec((NC, BM, FH), lambda i: (0, i, 0)),
            pl.BlockSpec((BM, 256), lambda i: (i, 0)),
        ],
        out_shape=[
            jax.ShapeDtypeStruct((NC, N, FH), jnp.float32),
            jax.ShapeDtypeStruct((N, 256), jnp.float32),
        ],
    )(x, wlT, wrT)


def _sc_segment_sum(pflat, src2, dst, z128, ones_h):
    mesh = plsc.VectorSubcoreMesh(core_axis_name="c", subcore_axis_name="s")

    @functools.partial(
        pl.kernel,
        mesh=mesh,
        out_type=jax.ShapeDtypeStruct((NC, NPAD, FH), jnp.float32),
        scratch_types=(
            pltpu.VMEM((2 * G, CHUNK), jnp.int32),
            pltpu.VMEM((2 * G, CHUNK), jnp.int32),
            pltpu.VMEM((2, CHUNK, FH), jnp.float32),
            pltpu.VMEM_SHARED((NPAD, FH), jnp.float32),
            pltpu.SemaphoreType.DMA,
            pltpu.SemaphoreType.DMA,
            pltpu.SemaphoreType.DMA,
        ),
    )
    def k(p_hbm, src_hbm, dst_hbm, z128_hbm,
          out_acc, isrc, idst, buf, acc, gsem, ssem, isem):
        c = lax.axis_index("c")
        s = lax.axis_index("s")
        pltpu.sync_copy(z128_hbm, acc.at[pl.ds(s * STRIPE, STRIPE)])
        pltpu.sync_copy(src_hbm.at[c].at[s].at[pl.ds(0, G)],
                        isrc.at[pl.ds(0, G)])
        pltpu.sync_copy(dst_hbm.at[s].at[pl.ds(0, G)], idst.at[pl.ds(0, G)])
        plsc.subcore_barrier()
        pltpu.async_copy(p_hbm.at[isrc.at[0]], buf.at[0], gsem)

        def body(j, carry):
            g = j // G
            gi = j - g * G
            slot = g % 2
            nj = j + 1
            pltpu.make_async_copy(
                p_hbm.at[isrc.at[0]], buf.at[0], gsem).wait()

            @pl.when(j >= 1)
            def _():
                pltpu.make_async_copy(
                    buf.at[0], acc.at[idst.at[0]], ssem).wait()

            @pl.when((gi == 0) & (g + 1 < NG))
            def _():
                off = (1 - slot) * G
                pltpu.async_copy(
                    src_hbm.at[c].at[s].at[pl.ds((g + 1) * G, G)],
                    isrc.at[pl.ds(off, G)], isem)
                pltpu.async_copy(
                    dst_hbm.at[s].at[pl.ds((g + 1) * G, G)],
                    idst.at[pl.ds(off, G)], isem)

            @pl.when((nj < CPT) & (nj % G == 0))
            def _():
                pltpu.make_async_copy(
                    src_hbm.at[c].at[s].at[pl.ds(0, G)],
                    isrc.at[pl.ds(0, G)], isem).wait()
                pltpu.make_async_copy(
                    dst_hbm.at[s].at[pl.ds(0, G)],
                    idst.at[pl.ds(0, G)], isem).wait()

            @pl.when(nj < CPT)
            def _():
                pltpu.async_copy(
                    p_hbm.at[isrc.at[((nj // G) % 2) * G + (nj - (nj // G) * G)]],
                    buf.at[nj % 2], gsem)

            pltpu.async_copy(buf.at[j % 2], acc.at[idst.at[slot * G + gi]],
                             ssem, add=True)
            return carry

        lax.fori_loop(0, CPT, body, 0)
        pltpu.make_async_copy(buf.at[0], acc.at[idst.at[0]], ssem).wait()
        plsc.subcore_barrier()
        sl = pl.ds(s * STRIPE, STRIPE)
        pltpu.sync_copy(acc.at[sl], out_acc.at[c].at[sl])

    @functools.partial(
        pl.kernel,
        mesh=mesh,
        out_type=jax.ShapeDtypeStruct((NC, NPAD, FH), jnp.float32),
        scratch_types=(
            pltpu.VMEM((CPT, CHUNK), jnp.int32),
            pltpu.VMEM((CHUNK, FH), jnp.float32),
            pltpu.VMEM_SHARED((NPAD, FH), jnp.float32),
            pltpu.SemaphoreType.DMA,
        ),
    )
    def kc(dst_hbm, z128_hbm, ones_hbm, out_cnt, idx_dst, ones_v, cnt, ssem):
        c = lax.axis_index("c")
        s = lax.axis_index("s")
        pltpu.sync_copy(z128_hbm, cnt.at[pl.ds(s * STRIPE, STRIPE)])
        pltpu.sync_copy(dst_hbm.at[s], idx_dst)
        pltpu.sync_copy(ones_hbm, ones_v)
        plsc.subcore_barrier()
        half = CPT // NC
        lo = c * half

        def body(j, carry):
            @pl.when(j >= lo + 4)
            def _():
                pltpu.make_async_copy(
                    ones_v, cnt.at[idx_dst.at[lo]], ssem).wait()

            pltpu.async_copy(ones_v, cnt.at[idx_dst.at[j]], ssem, add=True)
            return carry

        lax.fori_loop(lo, lo + half, body, 0)

        def drain(j, carry):
            pltpu.make_async_copy(ones_v, cnt.at[idx_dst.at[lo]], ssem).wait()
            return carry

        lax.fori_loop(0, 4, drain, 0)
        plsc.subcore_barrier()
        sl = pl.ds(s * STRIPE, STRIPE)
        pltpu.sync_copy(cnt.at[sl], out_cnt.at[c].at[sl])

    return k(pflat, src2, dst, z128), kc(dst, z128, ones_h)


def _mlp_body(acc_ref, cnt_ref, r_ref, bl_ref, waT_ref, ba_ref, w1T_ref,
              b1_ref, w2T_ref, b2_ref, w3T_ref, b3_ref, u_ref, v_ref):
    inv = 1.0 / jnp.maximum(cnt_ref[0, :, 0:1] + cnt_ref[1, :, 0:1], 1.0)
    h = jnp.concatenate([acc_ref[0] * inv, acc_ref[1] * inv], axis=1)
    h = jnp.maximum(h + bl_ref[...] + r_ref[...], 0.0)
    h = jnp.maximum(_dot(h, waT_ref[...]) + ba_ref[...], 0.0)
    h = jnp.maximum(_dot(h, w1T_ref[...]) + b1_ref[...], 0.0)
    h = jnp.maximum(_dot(h, w2T_ref[...]) + b2_ref[...], 0.0)
    z = _dot(h, w3T_ref[...]) + b3_ref[...]
    zc = z[:, :3]
    sq = jnp.sum(zc * zc, axis=1, keepdims=True)
    zh = zc.astype(jnp.bfloat16).astype(jnp.float32)
    zl = zc - zh
    sqh = sq.astype(jnp.bfloat16).astype(jnp.float32)
    sql = sq - sqh
    one = jnp.ones_like(sq)
    zero3 = jnp.zeros_like(zc)
    u_ref[...] = jnp.concatenate(
        [zh, zh, zl, sqh, sql, one, one, zero3], axis=1)
    v_ref[...] = jnp.concatenate(
        [-2.0 * zh, -2.0 * zl, -2.0 * zh, one, one, sqh, sql, zero3], axis=1)


def _mlp(acc, cnt, r, blv, waT, bav, w1T, b1v, w2T, b2v, w3T8, b3v):
    full = lambda shape: pl.BlockSpec(shape, lambda i: tuple(0 for _ in shape))
    return pl.pallas_call(
        _mlp_body,
        grid=(N // BM,),
        in_specs=[
            pl.BlockSpec((NC, BM, FH), lambda i: (0, i, 0)),
            pl.BlockSpec((NC, BM, FH), lambda i: (0, i, 0)),
            pl.BlockSpec((BM, 256), lambda i: (i, 0)),
            full((1, 256)),
            full((256, 128)), full((1, 128)),
            full((128, 64)), full((1, 64)),
            full((64, 32)), full((1, 32)),
            full((32, 8)), full((1, 8)),
        ],
        out_specs=[
            pl.BlockSpec((BM, 16), lambda i: (i, 0)),
            pl.BlockSpec((BM, 16), lambda i: (i, 0)),
        ],
        out_shape=[
            jax.ShapeDtypeStruct((N, 16), jnp.float32),
            jax.ShapeDtypeStruct((N, 16), jnp.float32),
        ],
    )(acc, cnt, r, blv, waT, bav, w1T, b1v, w2T, b2v, w3T8, b3v)


def _cdist_body(u_ref, v_ref, o_ref):
    d2 = lax.dot_general(u_ref[...], v_ref[...],
                         (((1,), (1,)), ((), ())),
                         preferred_element_type=jnp.float32)
    o_ref[...] = jnp.sqrt(jnp.maximum(d2, 1e-24))


def _cdist(u, v):
    return pl.pallas_call(
        _cdist_body,
        grid=(N // CB_M, pl.cdiv(N, CB_N)),
        in_specs=[
            pl.BlockSpec((CB_M, 16), lambda i, j: (i, 0)),
            pl.BlockSpec((CB_N, 16), lambda i, j: (j, 0)),
        ],
        out_specs=pl.BlockSpec((CB_M, CB_N), lambda i, j: (i, j)),
        out_shape=jax.ShapeDtypeStruct((N, N), jnp.float32),
    )(u, v)


def kernel(x, edge_index, Wl, bl, Wr, Wa, ba, W1, b1, W2, b2, W3, b3):
    ei = edge_index.astype(jnp.int32)
    src = ei[0]
    dst = ei[1]
    npd = EPAD - E
    pad_src = jnp.arange(npd, dtype=jnp.int32) % N
    pad_dst = N + (jnp.arange(npd, dtype=jnp.int32) % (NPAD - N))
    srcp = jnp.concatenate([src, pad_src]).reshape(NT, CPT, CHUNK)
    dstp = jnp.concatenate([dst, pad_dst]).reshape(NT, CPT, CHUNK)
    src2 = jnp.stack([srcp, srcp + N])

    p, r = _pre(x, Wl.T, Wr.T)
    pflat = p.reshape(NC * N, FH)

    z128 = jnp.zeros((STRIPE, FH), jnp.float32)
    ones_h = jnp.ones((CHUNK, FH), jnp.float32)
    acc, cnt = _sc_segment_sum(pflat, src2, dstp, z128, ones_h)

    blv = bl.reshape(1, 256)
    bav = ba.reshape(1, 128)
    b1v = b1.reshape(1, 64)
    b2v = b2.reshape(1, 32)
    w3T8 = jnp.concatenate([W3.T, jnp.zeros((32, 5), jnp.float32)], axis=1)
    b3v = jnp.concatenate([b3, jnp.zeros((5,), jnp.float32)]).reshape(1, 8)
    u, v = _mlp(acc, cnt, r,
                blv, Wa.T, bav, W1.T, b1v, W2.T, b2v, w3T8, b3v)
    return _cdist(u, v)


# speedup vs baseline: 5.9737x; 1.0403x over previous
import functools

import jax
import jax.numpy as jnp
from jax import lax
from jax.experimental import pallas as pl
from jax.experimental.pallas import tpu as pltpu
from jax.experimental.pallas import tpu_sc as plsc

N = 10000
E = 160000
FH = 128
NC = 2
NT = 16
CHUNK = 128
CPT = 80
G = 16
NG = CPT // G
EPAD = NT * CPT * CHUNK
NPAD = 10240
STRIPE = NPAD // NT

BM = 2000
CB_M = 2000
CB_N = 2048


def _dot(a, b, precision=None):
    return jnp.dot(a, b, preferred_element_type=jnp.float32,
                   precision=precision)


def _pre_body(x_ref, wlT_ref, wrT_ref, p_ref, r_ref):
    xb = x_ref[...]
    y = _dot(xb, wlT_ref[...], precision=lax.Precision.HIGHEST)
    p_ref[0] = y[:, :FH]
    p_ref[1] = y[:, FH:]
    r_ref[...] = _dot(xb, wrT_ref[...])


def _pre(x, wlT, wrT):
    return pl.pallas_call(
        _pre_body,
        grid=(N // BM,),
        in_specs=[
            pl.BlockSpec((BM, 512), lambda i: (i, 0)),
            pl.BlockSpec((512, 256), lambda i: (0, 0)),
            pl.BlockSpec((512, 256), lambda i: (0, 0)),
        ],
        out_specs=[
            pl.BlockSpec((NC, BM, FH), lambda i: (0, i, 0)),
            pl.BlockSpec((BM, 256), lambda i: (i, 0)),
        ],
        out_shape=[
            jax.ShapeDtypeStruct((NC, N, FH), jnp.float32),
            jax.ShapeDtypeStruct((N, 256), jnp.float32),
        ],
    )(x, wlT, wrT)


def _sc_segment_sum(pflat, src2, dst, z128, ones_h):
    mesh = plsc.VectorSubcoreMesh(core_axis_name="c", subcore_axis_name="s")

    @functools.partial(
        pl.kernel,
        mesh=mesh,
        out_type=jax.ShapeDtypeStruct((NC, NPAD, FH), jnp.float32),
        scratch_types=(
            pltpu.VMEM((2 * G, CHUNK), jnp.int32),
            pltpu.VMEM((2 * G, CHUNK), jnp.int32),
            pltpu.VMEM((2, CHUNK, FH), jnp.float32),
            pltpu.VMEM_SHARED((NPAD, FH), jnp.float32),
            pltpu.SemaphoreType.DMA,
            pltpu.SemaphoreType.DMA,
            pltpu.SemaphoreType.DMA,
        ),
    )
    def k(p_hbm, src_hbm, dst_hbm, z128_hbm,
          out_acc, isrc, idst, buf, acc, gsem, ssem, isem):
        c = lax.axis_index("c")
        s = lax.axis_index("s")
        pltpu.sync_copy(z128_hbm, acc.at[pl.ds(s * STRIPE, STRIPE)])
        pltpu.sync_copy(src_hbm.at[c].at[s].at[pl.ds(0, G)],
                        isrc.at[pl.ds(0, G)])
        pltpu.sync_copy(dst_hbm.at[s].at[pl.ds(0, G)], idst.at[pl.ds(0, G)])
        plsc.subcore_barrier()
        pltpu.async_copy(p_hbm.at[isrc.at[0]], buf.at[0], gsem)

        def body(j, carry):
            g = j // G
            gi = j - g * G
            slot = g % 2
            nj = j + 1
            pltpu.make_async_copy(
                p_hbm.at[isrc.at[0]], buf.at[0], gsem).wait()

            @pl.when(j >= 1)
            def _():
                pltpu.make_async_copy(
                    buf.at[0], acc.at[idst.at[0]], ssem).wait()

            @pl.when((gi == 0) & (g + 1 < NG))
            def _():
                off = (1 - slot) * G
                pltpu.async_copy(
                    src_hbm.at[c].at[s].at[pl.ds((g + 1) * G, G)],
                    isrc.at[pl.ds(off, G)], isem)
                pltpu.async_copy(
                    dst_hbm.at[s].at[pl.ds((g + 1) * G, G)],
                    idst.at[pl.ds(off, G)], isem)

            @pl.when((nj < CPT) & (nj % G == 0))
            def _():
                pltpu.make_async_copy(
                    src_hbm.at[c].at[s].at[pl.ds(0, G)],
                    isrc.at[pl.ds(0, G)], isem).wait()
                pltpu.make_async_copy(
                    dst_hbm.at[s].at[pl.ds(0, G)],
                    idst.at[pl.ds(0, G)], isem).wait()

            @pl.when(nj < CPT)
            def _():
                pltpu.async_copy(
                    p_hbm.at[isrc.at[((nj // G) % 2) * G + (nj - (nj // G) * G)]],
                    buf.at[nj % 2], gsem)

            pltpu.async_copy(buf.at[j % 2], acc.at[idst.at[slot * G + gi]],
                             ssem, add=True)
            return carry

        lax.fori_loop(0, CPT, body, 0)
        pltpu.make_async_copy(buf.at[0], acc.at[idst.at[0]], ssem).wait()
        plsc.subcore_barrier()
        sl = pl.ds(s * STRIPE, STRIPE)
        pltpu.sync_copy(acc.at[sl], out_acc.at[c].at[sl])

    @functools.partial(
        pl.kernel,
        mesh=mesh,
        out_type=jax.ShapeDtypeStruct((NC, NPAD, FH), jnp.float32),
        scratch_types=(
            pltpu.VMEM((CPT, CHUNK), jnp.int32),
            pltpu.VMEM((CHUNK, FH), jnp.float32),
            pltpu.VMEM_SHARED((NPAD, FH), jnp.float32),
            pltpu.SemaphoreType.DMA,
        ),
    )
    def kc(dst_hbm, z128_hbm, ones_hbm, out_cnt, idx_dst, ones_v, cnt, ssem):
        c = lax.axis_index("c")
        s = lax.axis_index("s")
        pltpu.sync_copy(z128_hbm, cnt.at[pl.ds(s * STRIPE, STRIPE)])
        pltpu.sync_copy(dst_hbm.at[s], idx_dst)
        pltpu.sync_copy(ones_hbm, ones_v)
        plsc.subcore_barrier()
        half = CPT // NC
        lo = c * half

        def body(j, carry):
            @pl.when(j >= lo + 4)
            def _():
                pltpu.make_async_copy(
                    ones_v, cnt.at[idx_dst.at[lo]], ssem).wait()

            pltpu.async_copy(ones_v, cnt.at[idx_dst.at[j]], ssem, add=True)
            return carry

        lax.fori_loop(lo, lo + half, body, 0)

        def drain(j, carry):
            pltpu.make_async_copy(ones_v, cnt.at[idx_dst.at[lo]], ssem).wait()
            return carry

        lax.fori_loop(0, 4, drain, 0)
        plsc.subcore_barrier()
        sl = pl.ds(s * STRIPE, STRIPE)
        pltpu.sync_copy(cnt.at[sl], out_cnt.at[c].at[sl])

    return k(pflat, src2, dst, z128), kc(dst, z128, ones_h)


def _mlp_body(acc_ref, cnt_ref, r_ref, bl_ref, waT_ref, ba_ref, w1T_ref,
              b1_ref, w2T_ref, b2_ref, w3T_ref, b3_ref, u_ref, v_ref):
    inv = 1.0 / jnp.maximum(cnt_ref[0, :, 0:1] + cnt_ref[1, :, 0:1], 1.0)
    h = jnp.concatenate([acc_ref[0] * inv, acc_ref[1] * inv], axis=1)
    h = jnp.maximum(h + bl_ref[...] + r_ref[...], 0.0)
    h = jnp.maximum(_dot(h, waT_ref[...]) + ba_ref[...], 0.0)
    h = jnp.maximum(_dot(h, w1T_ref[...]) + b1_ref[...], 0.0)
    h = jnp.maximum(_dot(h, w2T_ref[...]) + b2_ref[...], 0.0)
    z = _dot(h, w3T_ref[...]) + b3_ref[...]
    zc = z[:, :3]
    sq = jnp.sum(zc * zc, axis=1, keepdims=True)
    zh = zc.astype(jnp.bfloat16).astype(jnp.float32)
    zl = zc - zh
    sqh = sq.astype(jnp.bfloat16).astype(jnp.float32)
    sql = sq - sqh
    one = jnp.ones_like(sq)
    zero3 = jnp.zeros_like(zc)
    u_ref[...] = jnp.concatenate(
        [zh, zh, zl, sqh, sql, one, one, zero3], axis=1)
    v_ref[...] = jnp.concatenate(
        [-2.0 * zh, -2.0 * zl, -2.0 * zh, one, one, sqh, sql, zero3], axis=1)


def _mlp(acc, cnt, r, blv, waT, bav, w1T, b1v, w2T, b2v, w3T8, b3v):
    full = lambda shape: pl.BlockSpec(shape, lambda i: tuple(0 for _ in shape))
    return pl.pallas_call(
        _mlp_body,
        grid=(N // BM,),
        in_specs=[
            pl.BlockSpec((NC, BM, FH), lambda i: (0, i, 0)),
            pl.BlockSpec((NC, BM, FH), lambda i: (0, i, 0)),
            pl.BlockSpec((BM, 256), lambda i: (i, 0)),
            full((1, 256)),
            full((256, 128)), full((1, 128)),
            full((128, 64)), full((1, 64)),
            full((64, 32)), full((1, 32)),
            full((32, 8)), full((1, 8)),
        ],
        out_specs=[
            pl.BlockSpec((BM, 16), lambda i: (i, 0)),
            pl.BlockSpec((BM, 16), lambda i: (i, 0)),
        ],
        out_shape=[
            jax.ShapeDtypeStruct((N, 16), jnp.float32),
            jax.ShapeDtypeStruct((N, 16), jnp.float32),
        ],
    )(acc, cnt, r, blv, waT, bav, w1T, b1v, w2T, b2v, w3T8, b3v)


def _cdist_body(u_ref, v_ref, o_ref):
    d2 = lax.dot_general(u_ref[...], v_ref[...],
                         (((1,), (1,)), ((), ())),
                         preferred_element_type=jnp.float32)
    o_ref[...] = jnp.sqrt(jnp.maximum(d2, 1e-24))


def _cdist(u, v):
    return pl.pallas_call(
        _cdist_body,
        grid=(N // CB_M, pl.cdiv(N, CB_N)),
        in_specs=[
            pl.BlockSpec((CB_M, 16), lambda i, j: (i, 0)),
            pl.BlockSpec((CB_N, 16), lambda i, j: (j, 0)),
        ],
        out_specs=pl.BlockSpec((CB_M, CB_N), lambda i, j: (i, j)),
        out_shape=jax.ShapeDtypeStruct((N, N), jnp.float32),
    )(u, v)


def kernel(x, edge_index, Wl, bl, Wr, Wa, ba, W1, b1, W2, b2, W3, b3):
    ei = edge_index.astype(jnp.int32)
    src = ei[0]
    dst = ei[1]
    npd = EPAD - E
    pad_src = jnp.arange(npd, dtype=jnp.int32) % N
    pad_dst = N + (jnp.arange(npd, dtype=jnp.int32) % (NPAD - N))
    srcp = jnp.concatenate([src, pad_src]).reshape(NT, CPT, CHUNK)
    dstp = jnp.concatenate([dst, pad_dst]).reshape(NT, CPT, CHUNK)
    src2 = jnp.stack([srcp, srcp + N])

    p, r = _pre(x, Wl.T, Wr.T)
    pflat = p.reshape(NC * N, FH)

    z128 = jnp.zeros((STRIPE, FH), jnp.float32)
    ones_h = jnp.ones((CHUNK, FH), jnp.float32)
    acc, cnt = _sc_segment_sum(pflat, src2, dstp, z128, ones_h)

    blv = bl.reshape(1, 256)
    bav = ba.reshape(1, 128)
    b1v = b1.reshape(1, 64)
    b2v = b2.reshape(1, 32)
    w3T8 = jnp.concatenate([W3.T, jnp.zeros((32, 5), jnp.float32)], axis=1)
    b3v = jnp.concatenate([b3, jnp.zeros((5,), jnp.float32)]).reshape(1, 8)
    u, v = _mlp(acc, cnt, r,
                blv, Wa.T, bav, W1.T, b1v, W2.T, b2v, w3T8, b3v)
    return _cdist(u, v)


# speedup vs baseline: 6.2777x; 1.0509x over previous
import functools

import jax
import jax.numpy as jnp
from jax import lax
from jax.experimental import pallas as pl
from jax.experimental.pallas import tpu as pltpu
from jax.experimental.pallas import tpu_sc as plsc

N = 10000
E = 160000
FH = 128
NC = 2
NT = 16
CHUNK = 64
CPT = 160
G = 32
NG = CPT // G
NBUF = 4
EPAD = NT * CPT * CHUNK
NPAD = 10240
STRIPE = NPAD // NT

BM = 2000
CB_M = 2000
CB_N = 2048


def _dot(a, b, precision=None):
    return jnp.dot(a, b, preferred_element_type=jnp.float32,
                   precision=precision)


def _pre_body(x_ref, wlT_ref, wrT_ref, p_ref, r_ref):
    xb = x_ref[...]
    y = _dot(xb, wlT_ref[...], precision=lax.Precision.HIGHEST)
    p_ref[0] = y[:, :FH]
    p_ref[1] = y[:, FH:]
    r_ref[...] = _dot(xb, wrT_ref[...])


def _pre(x, wlT, wrT):
    return pl.pallas_call(
        _pre_body,
        grid=(N // BM,),
        in_specs=[
            pl.BlockSpec((BM, 512), lambda i: (i, 0)),
            pl.BlockSpec((512, 256), lambda i: (0, 0)),
            pl.BlockSpec((512, 256), lambda i: (0, 0)),
        ],
        out_specs=[
            pl.BlockSpec((NC, BM, FH), lambda i: (0, i, 0)),
            pl.BlockSpec((BM, 256), lambda i: (i, 0)),
        ],
        out_shape=[
            jax.ShapeDtypeStruct((NC, N, FH), jnp.float32),
            jax.ShapeDtypeStruct((N, 256), jnp.float32),
        ],
    )(x, wlT, wrT)


def _sc_segment_sum(pflat, src2, dst, z128, ones_h):
    mesh = plsc.VectorSubcoreMesh(core_axis_name="c", subcore_axis_name="s")

    @functools.partial(
        pl.kernel,
        mesh=mesh,
        out_type=jax.ShapeDtypeStruct((NC, NPAD, FH), jnp.float32),
        scratch_types=(
            pltpu.VMEM((2 * G, CHUNK), jnp.int32),
            pltpu.VMEM((2 * G, CHUNK), jnp.int32),
            pltpu.VMEM((NBUF, CHUNK, FH), jnp.float32),
            pltpu.VMEM_SHARED((NPAD, FH), jnp.float32),
            pltpu.SemaphoreType.DMA,
            pltpu.SemaphoreType.DMA,
            pltpu.SemaphoreType.DMA,
        ),
    )
    def k(p_hbm, src_hbm, dst_hbm, z128_hbm,
          out_acc, isrc, idst, buf, acc, gsem, ssem, isem):
        c = lax.axis_index("c")
        s = lax.axis_index("s")
        pltpu.sync_copy(z128_hbm, acc.at[pl.ds(s * STRIPE, STRIPE)])
        pltpu.sync_copy(src_hbm.at[c].at[s].at[pl.ds(0, G)],
                        isrc.at[pl.ds(0, G)])
        pltpu.sync_copy(dst_hbm.at[s].at[pl.ds(0, G)], idst.at[pl.ds(0, G)])
        plsc.subcore_barrier()
        pltpu.async_copy(p_hbm.at[isrc.at[0]], buf.at[0], gsem)
        pltpu.async_copy(p_hbm.at[isrc.at[1]], buf.at[1], gsem)

        def body(j, carry):
            g = j // G
            gi = j - g * G
            slot = g % 2
            nj = j + 2
            pltpu.make_async_copy(
                p_hbm.at[isrc.at[0]], buf.at[0], gsem).wait()

            @pl.when(j >= 2)
            def _():
                pltpu.make_async_copy(
                    buf.at[0], acc.at[idst.at[0]], ssem).wait()

            @pl.when((gi == 1) & (g + 1 < NG))
            def _():
                off = (1 - slot) * G
                pltpu.async_copy(
                    src_hbm.at[c].at[s].at[pl.ds((g + 1) * G, G)],
                    isrc.at[pl.ds(off, G)], isem)
                pltpu.async_copy(
                    dst_hbm.at[s].at[pl.ds((g + 1) * G, G)],
                    idst.at[pl.ds(off, G)], isem)

            @pl.when((nj < CPT) & (gi == G - 2))
            def _():
                pltpu.make_async_copy(
                    src_hbm.at[c].at[s].at[pl.ds(0, G)],
                    isrc.at[pl.ds(0, G)], isem).wait()
                pltpu.make_async_copy(
                    dst_hbm.at[s].at[pl.ds(0, G)],
                    idst.at[pl.ds(0, G)], isem).wait()

            @pl.when(nj < CPT)
            def _():
                pltpu.async_copy(
                    p_hbm.at[isrc.at[((nj // G) % 2) * G + (nj - (nj // G) * G)]],
                    buf.at[nj % NBUF], gsem)

            pltpu.async_copy(buf.at[j % NBUF], acc.at[idst.at[slot * G + gi]],
                             ssem, add=True)
            return carry

        lax.fori_loop(0, CPT, body, 0)
        pltpu.make_async_copy(buf.at[0], acc.at[idst.at[0]], ssem).wait()
        pltpu.make_async_copy(buf.at[0], acc.at[idst.at[0]], ssem).wait()
        plsc.subcore_barrier()
        sl = pl.ds(s * STRIPE, STRIPE)
        pltpu.sync_copy(acc.at[sl], out_acc.at[c].at[sl])

    @functools.partial(
        pl.kernel,
        mesh=mesh,
        out_type=jax.ShapeDtypeStruct((NC, NPAD, FH), jnp.float32),
        scratch_types=(
            pltpu.VMEM((CPT, CHUNK), jnp.int32),
            pltpu.VMEM((CHUNK, FH), jnp.float32),
            pltpu.VMEM_SHARED((NPAD, FH), jnp.float32),
            pltpu.SemaphoreType.DMA,
        ),
    )
    def kc(dst_hbm, z128_hbm, ones_hbm, out_cnt, idx_dst, ones_v, cnt, ssem):
        c = lax.axis_index("c")
        s = lax.axis_index("s")
        pltpu.sync_copy(z128_hbm, cnt.at[pl.ds(s * STRIPE, STRIPE)])
        pltpu.sync_copy(dst_hbm.at[s], idx_dst)
        pltpu.sync_copy(ones_hbm, ones_v)
        plsc.subcore_barrier()
        half = CPT // NC
        lo = c * half

        def body(j, carry):
            @pl.when(j >= lo + 4)
            def _():
                pltpu.make_async_copy(
                    ones_v, cnt.at[idx_dst.at[lo]], ssem).wait()

            pltpu.async_copy(ones_v, cnt.at[idx_dst.at[j]], ssem, add=True)
            return carry

        lax.fori_loop(lo, lo + half, body, 0)

        def drain(j, carry):
            pltpu.make_async_copy(ones_v, cnt.at[idx_dst.at[lo]], ssem).wait()
            return carry

        lax.fori_loop(0, 4, drain, 0)
        plsc.subcore_barrier()
        sl = pl.ds(s * STRIPE, STRIPE)
        pltpu.sync_copy(cnt.at[sl], out_cnt.at[c].at[sl])

    return k(pflat, src2, dst, z128), kc(dst, z128, ones_h)


def _mlp_body(acc_ref, cnt_ref, r_ref, bl_ref, waT_ref, ba_ref, w1T_ref,
              b1_ref, w2T_ref, b2_ref, w3T_ref, b3_ref, u_ref, v_ref):
    inv = 1.0 / jnp.maximum(cnt_ref[0, :, 0:1] + cnt_ref[1, :, 0:1], 1.0)
    h = jnp.concatenate([acc_ref[0] * inv, acc_ref[1] * inv], axis=1)
    h = jnp.maximum(h + bl_ref[...] + r_ref[...], 0.0)
    h = jnp.maximum(_dot(h, waT_ref[...]) + ba_ref[...], 0.0)
    h = jnp.maximum(_dot(h, w1T_ref[...]) + b1_ref[...], 0.0)
    h = jnp.maximum(_dot(h, w2T_ref[...]) + b2_ref[...], 0.0)
    z = _dot(h, w3T_ref[...]) + b3_ref[...]
    zc = z[:, :3]
    sq = jnp.sum(zc * zc, axis=1, keepdims=True)
    zh = zc.astype(jnp.bfloat16).astype(jnp.float32)
    zl = zc - zh
    sqh = sq.astype(jnp.bfloat16).astype(jnp.float32)
    sql = sq - sqh
    one = jnp.ones_like(sq)
    zero3 = jnp.zeros_like(zc)
    u_ref[...] = jnp.concatenate(
        [zh, zh, zl, sqh, sql, one, one, zero3], axis=1)
    v_ref[...] = jnp.concatenate(
        [-2.0 * zh, -2.0 * zl, -2.0 * zh, one, one, sqh, sql, zero3], axis=1)


def _mlp(acc, cnt, r, blv, waT, bav, w1T, b1v, w2T, b2v, w3T8, b3v):
    full = lambda shape: pl.BlockSpec(shape, lambda i: tuple(0 for _ in shape))
    return pl.pallas_call(
        _mlp_body,
        grid=(N // BM,),
        in_specs=[
            pl.BlockSpec((NC, BM, FH), lambda i: (0, i, 0)),
            pl.BlockSpec((NC, BM, FH), lambda i: (0, i, 0)),
            pl.BlockSpec((BM, 256), lambda i: (i, 0)),
            full((1, 256)),
            full((256, 128)), full((1, 128)),
            full((128, 64)), full((1, 64)),
            full((64, 32)), full((1, 32)),
            full((32, 8)), full((1, 8)),
        ],
        out_specs=[
            pl.BlockSpec((BM, 16), lambda i: (i, 0)),
            pl.BlockSpec((BM, 16), lambda i: (i, 0)),
        ],
        out_shape=[
            jax.ShapeDtypeStruct((N, 16), jnp.float32),
            jax.ShapeDtypeStruct((N, 16), jnp.float32),
        ],
    )(acc, cnt, r, blv, waT, bav, w1T, b1v, w2T, b2v, w3T8, b3v)


def _cdist_body(u_ref, v_ref, o_ref):
    d2 = lax.dot_general(u_ref[...], v_ref[...],
                         (((1,), (1,)), ((), ())),
                         preferred_element_type=jnp.float32)
    o_ref[...] = jnp.sqrt(jnp.maximum(d2, 1e-24))


def _cdist(u, v):
    return pl.pallas_call(
        _cdist_body,
        grid=(N // CB_M, pl.cdiv(N, CB_N)),
        in_specs=[
            pl.BlockSpec((CB_M, 16), lambda i, j: (i, 0)),
            pl.BlockSpec((CB_N, 16), lambda i, j: (j, 0)),
        ],
        out_specs=pl.BlockSpec((CB_M, CB_N), lambda i, j: (i, j)),
        out_shape=jax.ShapeDtypeStruct((N, N), jnp.float32),
    )(u, v)


def kernel(x, edge_index, Wl, bl, Wr, Wa, ba, W1, b1, W2, b2, W3, b3):
    ei = edge_index.astype(jnp.int32)
    src = ei[0]
    dst = ei[1]
    npd = EPAD - E
    pad_src = jnp.arange(npd, dtype=jnp.int32) % N
    pad_dst = N + (jnp.arange(npd, dtype=jnp.int32) % (NPAD - N))
    srcp = jnp.concatenate([src, pad_src]).reshape(NT, CPT, CHUNK)
    dstp = jnp.concatenate([dst, pad_dst]).reshape(NT, CPT, CHUNK)
    src2 = jnp.stack([srcp, srcp + N])

    p, r = _pre(x, Wl.T, Wr.T)
    pflat = p.reshape(NC * N, FH)

    z128 = jnp.zeros((STRIPE, FH), jnp.float32)
    ones_h = jnp.ones((CHUNK, FH), jnp.float32)
    acc, cnt = _sc_segment_sum(pflat, src2, dstp, z128, ones_h)

    blv = bl.reshape(1, 256)
    bav = ba.reshape(1, 128)
    b1v = b1.reshape(1, 64)
    b2v = b2.reshape(1, 32)
    w3T8 = jnp.concatenate([W3.T, jnp.zeros((32, 5), jnp.float32)], axis=1)
    b3v = jnp.concatenate([b3, jnp.zeros((5,), jnp.float32)]).reshape(1, 8)
    u, v = _mlp(acc, cnt, r,
                blv, Wa.T, bav, W1.T, b1v, W2.T, b2v, w3T8, b3v)
    return _cdist(u, v)
